# baseline (speedup 1.0000x reference)
import functools

import jax
import jax.numpy as jnp
from jax import lax
from jax.experimental import pallas as pl
from jax.experimental.pallas import tpu as pltpu
from jax.experimental.pallas import tpu_sc as plsc

_B = 16384
_D = 64
_NC = 2
_NS = 16
_NW = _NC * _NS
_BPW = _B // _NW
_GCHUNK = 128
_NCHUNK = _BPW // _GCHUNK
_EPS2 = 1e-24
_SCALE = 1.0 / _B
_TCBLK = 2048


@functools.partial(
    pl.kernel,
    mesh=plsc.VectorSubcoreMesh(core_axis_name="c", subcore_axis_name="s"),
    out_type=jax.ShapeDtypeStruct((_B, _D), jnp.float32),
    scratch_types=[
        pltpu.VMEM((_NCHUNK, _GCHUNK), jnp.int32),
        pltpu.VMEM((_BPW, _D), jnp.float32),
        pltpu.SemaphoreType.DMA,
    ],
    compiler_params=pltpu.CompilerParams(use_tc_tiling_on_sc=False),
)
def _gather_sc(labels_hbm, centers_hbm, out_hbm, idx_v, rows_v, sem):
    wid = lax.axis_index("s") * _NC + lax.axis_index("c")
    base = wid * _BPW
    pltpu.sync_copy(labels_hbm.at[pl.ds(wid * _NCHUNK, _NCHUNK)], idx_v)
    cps = [
        pltpu.async_copy(centers_hbm.at[idx_v.at[j]],
                         rows_v.at[pl.ds(j * _GCHUNK, _GCHUNK)], sem)
        for j in range(_NCHUNK)
    ]
    for cp in cps:
        cp.wait()
    pltpu.sync_copy(rows_v, out_hbm.at[pl.ds(base, _BPW)])


def _loss_tc_body(f_ref, c_ref, o_ref):
    f = f_ref[...]
    c = c_ref[...]
    sf = jnp.sum(f * f, axis=1)
    sc = jnp.sum(c * c, axis=1)
    fc = jnp.sum(f * c, axis=1)
    rf = lax.rsqrt(jnp.maximum(sf, _EPS2))
    rc = lax.rsqrt(jnp.maximum(sc, _EPS2))
    part = jnp.sum(sf * rf * rf + sc * rc * rc - 2.0 * (fc * rf) * rc)

    @pl.when(pl.program_id(0) == 0)
    def _():
        o_ref[0, 0] = 0.0

    o_ref[0, 0] += part * _SCALE


_loss_tc = pl.pallas_call(
    _loss_tc_body,
    grid=(_B // _TCBLK,),
    in_specs=[
        pl.BlockSpec((_TCBLK, _D), lambda i: (i, 0)),
        pl.BlockSpec((_TCBLK, _D), lambda i: (i, 0)),
    ],
    out_specs=pl.BlockSpec(memory_space=pltpu.SMEM),
    out_shape=jax.ShapeDtypeStruct((1, 1), jnp.float32),
)


def kernel(feats, labels, centers):
    labels2d = labels.astype(jnp.int32).reshape(_B // _GCHUNK, _GCHUNK)
    rows = _gather_sc(labels2d, centers)
    loss = _loss_tc(feats, rows)
    return loss[0, 0]


# speedup vs baseline: 2.3795x; 2.3795x over previous
import functools

import jax
import jax.numpy as jnp
from jax import lax
from jax.experimental import pallas as pl
from jax.experimental.pallas import tpu as pltpu
from jax.experimental.pallas import tpu_sc as plsc

_B = 16384
_D = 64
_NCLS = 1000000
_TROWS = 8
_NTILE = _NCLS // _TROWS
_NC = 2
_NS = 16
_NW = _NC * _NS
_BPW = _B // _NW
_L = 16
_RING = 32
_LROWS = 4
_EPS2 = 1e-24
_SCALE = 1.0 / _B
_TCBLK = 2048


@functools.partial(
    pl.kernel,
    mesh=plsc.VectorSubcoreMesh(core_axis_name="c", subcore_axis_name="s"),
    out_type=jax.ShapeDtypeStruct((_B, _D), jnp.float32),
    scratch_types=[
        pltpu.VMEM((_LROWS, 128), jnp.int32),
        pltpu.VMEM((_BPW, _D), jnp.float32),
        pltpu.SemaphoreType.DMA,
    ],
)
def _gather_sc(labels_hbm, centers_hbm, out_hbm, lab_v, comp_v, sem):
    wid = lax.axis_index("s") * _NC + lax.axis_index("c")
    base = wid * _BPW
    pltpu.sync_copy(labels_hbm.at[pl.ds(wid * _LROWS, _LROWS)], lab_v)

    cps = []
    for i in range(_BPW // _L):
        v = lab_v[i // 8, pl.ds((i % 8) * _L, _L)]
        tv = v >> 3
        rv = v & 7
        for k in range(_L):
            g = i * _L + k
            if len(cps) >= _RING:
                cps[g - _RING].wait()
            cps.append(
                pltpu.async_copy(centers_hbm.at[tv[k], rv[k]],
                                 comp_v.at[g], sem))
    for cp in cps[-_RING:]:
        cp.wait()

    pltpu.sync_copy(comp_v, out_hbm.at[pl.ds(base, _BPW)])


def _loss_tc_body(f_ref, c_ref, o_ref):
    f = f_ref[...]
    c = c_ref[...]
    sf = jnp.sum(f * f, axis=1)
    sc = jnp.sum(c * c, axis=1)
    fc = jnp.sum(f * c, axis=1)
    rf = lax.rsqrt(jnp.maximum(sf, _EPS2))
    rc = lax.rsqrt(jnp.maximum(sc, _EPS2))
    part = jnp.sum(sf * rf * rf + sc * rc * rc - 2.0 * (fc * rf) * rc)

    @pl.when(pl.program_id(0) == 0)
    def _():
        o_ref[0, 0] = 0.0

    o_ref[0, 0] += part * _SCALE


_loss_tc = pl.pallas_call(
    _loss_tc_body,
    grid=(_B // _TCBLK,),
    in_specs=[
        pl.BlockSpec((_TCBLK, _D), lambda i: (i, 0)),
        pl.BlockSpec((_TCBLK, _D), lambda i: (i, 0)),
    ],
    out_specs=pl.BlockSpec(memory_space=pltpu.SMEM),
    out_shape=jax.ShapeDtypeStruct((1, 1), jnp.float32),
)


def kernel(feats, labels, centers):
    labels2d = labels.astype(jnp.int32).reshape(_B // 128, 128)
    centers3 = centers.reshape(_NTILE, _TROWS, _D)
    rows = _gather_sc(labels2d, centers3)
    loss = _loss_tc(feats, rows)
    return loss[0, 0]
